# Initial kernel scaffold; baseline (speedup 1.0000x reference)
import jax
import jax.numpy as jnp
from jax.experimental import pallas as pl


def kernel(species, coordinates, net_charge, W_aev, eW1, eb1, eW2, eb2, eW3, eb3, eW4, eb4, sW1, sb1, sW2, sb2, sW3, sb3):
    raise NotImplementedError("write your pallas kernel here")


# speedup vs baseline: 1.1518x; 1.1518x over previous
import jax
import jax.numpy as jnp
from jax.experimental import pallas as pl
from jax.experimental.pallas import tpu as pltpu

B, A, L, OUT_DIM, E = 1024, 48, 384, 16, 4
N = B * A
TILE = 512
NT = N // TILE


def _celu(x, alpha):
    return jnp.where(x > 0, x, alpha * (jnp.exp(x / alpha) - 1.0))


def _atoms_kernel(coords_ref, species_ref, Waev_ref,
                  eW1, eb1, eW2, eb2, eW3, eb3, eW4, eb4,
                  out_ref):
    c = coords_ref[...]
    aev = jnp.tanh(jax.lax.dot(c, Waev_ref[...],
                               preferred_element_type=jnp.float32))
    sp = species_ref[...]
    acc = jnp.zeros((TILE, OUT_DIM), jnp.float32)
    for e in range(E):
        h = _celu(jax.lax.dot(aev, eW1[e], preferred_element_type=jnp.float32)
                  + eb1[e], 0.1)
        h = _celu(jax.lax.dot(h, eW2[e], preferred_element_type=jnp.float32)
                  + eb2[e], 0.1)
        h = _celu(jax.lax.dot(h, eW3[e], preferred_element_type=jnp.float32)
                  + eb3[e], 0.1)
        h = jax.lax.dot(h, eW4[e], preferred_element_type=jnp.float32) + eb4[e]
        acc = jnp.where(sp == e, h, acc)
    out_ref[...] = acc


def _mol_kernel(out3d_ref, xs_ref, ys_ref, zs_ref, charge_ref,
                sW1, sb1, sW2, sb2, sW3, sb3,
                en_ref):
    s = out3d_ref[:, 0, :]
    for a in range(1, A):
        s = s + out3d_ref[:, a, :]
    mean = s * (1.0 / A)

    xs = xs_ref[...]
    ys = ys_ref[...]
    zs = zs_ref[...]
    inv_a = 1.0 / A
    cx = jnp.sum(xs, axis=1, keepdims=True) * inv_a
    cy = jnp.sum(ys, axis=1, keepdims=True) * inv_a
    cz = jnp.sum(zs, axis=1, keepdims=True) * inv_a
    dist = jnp.sqrt((xs - cx) ** 2 + (ys - cy) ** 2 + (zs - cz) ** 2)
    sum_dist = jnp.sum(dist, axis=1, keepdims=True)
    mean_dist = sum_dist * inv_a
    max_dist = jnp.max(dist, axis=1, keepdims=True)
    smoothmax = jnp.log(jnp.sum(jnp.exp(dist - max_dist), axis=1,
                                keepdims=True)) + max_dist

    mf = jnp.concatenate(
        [s, mean, sum_dist, mean_dist, smoothmax, charge_ref[...]], axis=1)
    h = _celu(jax.lax.dot(mf, sW1[...], preferred_element_type=jnp.float32)
              + sb1[...], 1.0)
    h = _celu(jax.lax.dot(h, sW2[...], preferred_element_type=jnp.float32)
              + sb2[...], 1.0)
    en = jax.lax.dot(h, sW3[...], preferred_element_type=jnp.float32) + sb3[...]
    en_ref[...] = en


def _full(shape):
    nd = len(shape)
    return pl.BlockSpec(shape, lambda *_: (0,) * nd)


def kernel(species, coordinates, net_charge, W_aev,
           eW1, eb1, eW2, eb2, eW3, eb3, eW4, eb4,
           sW1, sb1, sW2, sb2, sW3, sb3):
    coords_flat = coordinates.reshape(N, 3)
    species2d = species.reshape(N, 1).astype(jnp.int32)
    eb = [b.reshape(E, 1, -1) for b in (eb1, eb2, eb3, eb4)]

    out = pl.pallas_call(
        _atoms_kernel,
        grid=(NT,),
        in_specs=[
            pl.BlockSpec((TILE, 3), lambda i: (i, 0)),
            pl.BlockSpec((TILE, 1), lambda i: (i, 0)),
            _full((3, L)),
            _full(eW1.shape), _full(eb[0].shape),
            _full(eW2.shape), _full(eb[1].shape),
            _full(eW3.shape), _full(eb[2].shape),
            _full(eW4.shape), _full(eb[3].shape),
        ],
        out_specs=pl.BlockSpec((TILE, OUT_DIM), lambda i: (i, 0)),
        out_shape=jax.ShapeDtypeStruct((N, OUT_DIM), jnp.float32),
        compiler_params=pltpu.CompilerParams(
            dimension_semantics=("arbitrary",)),
    )(coords_flat, species2d, W_aev,
      eW1, eb[0], eW2, eb[1], eW3, eb[2], eW4, eb[3])

    out3d = out.reshape(B, A, OUT_DIM)
    xs = coordinates[:, :, 0]
    ys = coordinates[:, :, 1]
    zs = coordinates[:, :, 2]
    sb = [b.reshape(1, -1) for b in (sb1, sb2, sb3)]

    en = pl.pallas_call(
        _mol_kernel,
        in_specs=[
            _full((B, A, OUT_DIM)),
            _full((B, A)), _full((B, A)), _full((B, A)),
            _full((B, 1)),
            _full(sW1.shape), _full(sb[0].shape),
            _full(sW2.shape), _full(sb[1].shape),
            _full(sW3.shape), _full(sb[2].shape),
        ],
        out_specs=_full((B, 1)),
        out_shape=jax.ShapeDtypeStruct((B, 1), jnp.float32),
    )(out3d, xs, ys, zs, net_charge.reshape(B, 1),
      sW1, sb[0], sW2, sb[1], sW3, sb[2])

    return (species, en[:, 0])
